# baseline (speedup 1.0000x reference)
import functools

import jax
import jax.numpy as jnp
from jax import lax
from jax.experimental import pallas as pl
from jax.experimental.pallas import tpu as pltpu
from jax.experimental.pallas import tpu_sc as plsc

F32 = jnp.float32
EPS = 1e-5

NC, NS = 2, 16
NW = NC * NS


def _node_proj_body(x_ref, w_ref, wf1_ref, v_ref, o_ref):
    h = jnp.dot(x_ref[...], w_ref[...], preferred_element_type=F32)
    h = h + v_ref[0:1, :]
    m = jnp.mean(h, axis=-1, keepdims=True)
    hc = h - m
    var = jnp.mean(hc * hc, axis=-1, keepdims=True)
    y = hc * lax.rsqrt(var + EPS) * v_ref[1:2, :] + v_ref[2:3, :]
    o_ref[...] = jnp.dot(y, wf1_ref[...], preferred_element_type=F32)


def _node_proj(node_feats, W_np, Wf1, b_np, g_np, be_np):
    n, k = node_feats.shape
    blk = 2000
    grid = n // blk
    vecs = jnp.concatenate(
        [b_np[None], g_np[None], be_np[None], jnp.zeros((5, 128), F32)], axis=0
    )
    return pl.pallas_call(
        _node_proj_body,
        grid=(grid,),
        in_specs=[
            pl.BlockSpec((blk, k), lambda i: (i, 0)),
            pl.BlockSpec((k, 128), lambda i: (0, 0)),
            pl.BlockSpec((128, 128), lambda i: (0, 0)),
            pl.BlockSpec((8, 128), lambda i: (0, 0)),
        ],
        out_specs=pl.BlockSpec((blk, 128), lambda i: (i, 0)),
        out_shape=jax.ShapeDtypeStruct((n, 128), F32),
    )(node_feats, W_np, Wf1, vecs)


def _cond_proj_body(c_ref, w1_ref, w2_ref, v_ref, b2_ref, o_ref):
    h = jnp.dot(c_ref[...], w1_ref[...], preferred_element_type=F32)
    h = h + v_ref[0:1, :]
    m = jnp.mean(h, axis=-1, keepdims=True)
    hc = h - m
    var = jnp.mean(hc * hc, axis=-1, keepdims=True)
    y = hc * lax.rsqrt(var + EPS) * v_ref[1:2, :] + v_ref[2:3, :]
    gb = jnp.dot(y, w2_ref[...], preferred_element_type=F32) + b2_ref[0:1, :]
    lane = lax.broadcasted_iota(jnp.int32, gb.shape, 1)
    o_ref[...] = gb + (lane < 128).astype(F32)


def _cond_proj(cond, Wc1, bc1, g_c, be_c, Wc2, bc2):
    vecs = jnp.concatenate(
        [bc1[None], g_c[None], be_c[None], jnp.zeros((5, 128), F32)], axis=0
    )
    b2 = jnp.concatenate([bc2[None], jnp.zeros((7, 256), F32)], axis=0)
    return pl.pallas_call(
        _cond_proj_body,
        grid=(1,),
        in_specs=[
            pl.BlockSpec((16, 128), lambda i: (0, 0)),
            pl.BlockSpec((128, 128), lambda i: (0, 0)),
            pl.BlockSpec((128, 256), lambda i: (0, 0)),
            pl.BlockSpec((8, 128), lambda i: (0, 0)),
            pl.BlockSpec((8, 256), lambda i: (0, 0)),
        ],
        out_specs=pl.BlockSpec((16, 256), lambda i: (0, 0)),
        out_shape=jax.ShapeDtypeStruct((16, 256), F32),
    )(cond, Wc1, Wc2, vecs, b2)


def _sc_gather_sum(nf2, src, dst, n_edges):
    C = 128
    n_groups = n_edges // C
    base_g, extra = divmod(n_groups, NW)

    mesh = plsc.VectorSubcoreMesh(
        core_axis_name="c", subcore_axis_name="s", num_cores=NC, num_subcores=NS
    )

    @functools.partial(
        pl.kernel,
        out_type=jax.ShapeDtypeStruct((n_edges, 128), F32),
        mesh=mesh,
        scratch_types=[
            pltpu.VMEM((C,), jnp.int32),
            pltpu.VMEM((C,), jnp.int32),
            pltpu.VMEM((C, 128), F32),
            pltpu.VMEM((C, 128), F32),
            pltpu.SemaphoreType.DMA,
            pltpu.SemaphoreType.DMA,
        ],
    )
    def sc_kernel(nf2_hbm, src_hbm, dst_hbm, out_hbm, idx_s, idx_d, rows_a,
                  rows_b, sem_a, sem_b):
        wid = lax.axis_index("s") * NC + lax.axis_index("c")
        g0 = wid * base_g + jnp.minimum(wid, extra)
        ng = base_g + jnp.where(wid < extra, 1, 0)

        def group_body(i, carry):
            base = (g0 + i) * C
            pltpu.sync_copy(src_hbm.at[pl.ds(base, C)], idx_s)
            pltpu.sync_copy(dst_hbm.at[pl.ds(base, C)], idx_d)
            cp_a = pltpu.async_copy(nf2_hbm.at[idx_s], rows_a, sem_a)
            cp_b = pltpu.async_copy(nf2_hbm.at[idx_d], rows_b, sem_b)
            cp_a.wait()
            cp_b.wait()

            def add_body(r, c2):
                for cc in range(8):
                    sl = pl.ds(cc * 16, 16)
                    rows_a[r, sl] = rows_a[r, sl] + rows_b[r, sl]
                return c2

            lax.fori_loop(0, C, add_body, 0)
            pltpu.sync_copy(rows_a, out_hbm.at[pl.ds(base, C)])
            return carry

        lax.fori_loop(0, ng, group_body, 0)

    return sc_kernel(nf2, src, dst)


def _edge_body(s_ref, xt_ref, ab_ref, cv_ref, bnd_ref, gbt_ref, o_ref, *,
               blk):
    xt = xt_ref[...]
    s_in = s_ref[...]
    c1 = cv_ref[0:1, :]
    vrow = cv_ref[1:2, :]
    crow = cv_ref[2:3, :]
    bbar = cv_ref[3, 0]
    ccst = cv_ref[3, 1]

    big = lax.dot_general(
        xt, ab_ref[...], ((( 0,), (0,)), ((), ())),
        preferred_element_type=F32)
    t = big[:, :128] + c1
    mu = big[:, 128:129] + bbar
    q = big[:, 129:137]
    xu = big[:, 137:138]
    varg = jnp.sum(q * q, axis=-1, keepdims=True) + 2.0 * xu + ccst
    inv_sg = lax.rsqrt(varg + EPS)

    y0 = s_in + (t - mu * vrow) * inv_sg + crow
    m = jnp.mean(y0, axis=-1, keepdims=True)
    yc = y0 - m
    var = jnp.mean(yc * yc, axis=-1, keepdims=True)
    y = yc * lax.rsqrt(var + EPS)

    i = pl.program_id(0)
    gidx = i * blk + lax.broadcasted_iota(jnp.int32, (blk, 1), 0)
    starts = bnd_ref[0:1, :]
    ends = bnd_ref[1:2, :]
    oh = jnp.logical_and(gidx >= starts, gidx < ends).astype(F32)
    gb = jnp.dot(oh, gbt_ref[...], preferred_element_type=F32)
    o_ref[...] = jnp.maximum(y * gb[:, :128] + gb[:, 128:], 0.0)


def _edge_fuse(S, geo_t, abig, cvec, bounds, gbt):
    n_edges = S.shape[0]
    blk = 1280
    grid = n_edges // blk
    return pl.pallas_call(
        functools.partial(_edge_body, blk=blk),
        grid=(grid,),
        in_specs=[
            pl.BlockSpec((blk, 128), lambda i: (i, 0)),
            pl.BlockSpec((8, blk), lambda i: (0, i)),
            pl.BlockSpec((8, 256), lambda i: (0, 0)),
            pl.BlockSpec((8, 128), lambda i: (0, 0)),
            pl.BlockSpec((8, 16), lambda i: (0, 0)),
            pl.BlockSpec((16, 256), lambda i: (0, 0)),
        ],
        out_specs=pl.BlockSpec((blk, 128), lambda i: (i, 0)),
        out_shape=jax.ShapeDtypeStruct((n_edges, 128), F32),
    )(S, geo_t, abig, cvec, bounds, gbt)


def kernel(node_feats, edge_index, edge_geo, cond, batch_ids,
           W_np, b_np, g_np, be_np,
           W_geo, b_geo, g_geo, be_geo,
           Wc1, bc1, g_c, be_c, Wc2, bc2,
           Wf, bf):
    n_edges = edge_index.shape[1]
    src = edge_index[0].astype(jnp.int32)
    dst = edge_index[1].astype(jnp.int32)
    geo_t = edge_geo.T
    bids = batch_ids.astype(jnp.int32)
    starts = jnp.searchsorted(bids, jnp.arange(16, dtype=jnp.int32)).astype(jnp.int32)
    ends = jnp.concatenate([starts[1:], jnp.array([n_edges], jnp.int32)])
    bounds = jnp.concatenate(
        [starts[None], ends[None], jnp.zeros((6, 16), jnp.int32)], axis=0)

    Wf1 = Wf[:128]
    Wf2 = Wf[128:]

    Wgsum = W_geo.reshape(8, 8, 128).sum(axis=0)
    A = Wgsum @ (g_geo[:, None] * Wf2)
    c1 = (b_geo * g_geo) @ Wf2
    vrow = g_geo @ Wf2
    crow = be_geo @ Wf2 + bf
    m8 = Wgsum.mean(axis=1)
    bbar = b_geo.mean()
    acen = Wgsum - m8[:, None]
    bcen = b_geo - bbar
    Mc = (acen @ acen.T) / 128.0
    uc = (acen @ bcen) / 128.0
    ccst = jnp.dot(bcen, bcen) / 128.0
    L = jnp.linalg.cholesky(Mc + 1e-12 * jnp.eye(8, dtype=F32))

    misc = jnp.zeros((128,), F32).at[0].set(bbar).at[1].set(ccst)
    cvec = jnp.concatenate(
        [c1[None], vrow[None], crow[None], misc[None], jnp.zeros((4, 128), F32)],
        axis=0,
    )
    abig = jnp.concatenate(
        [A, m8[:, None], L, uc[:, None], jnp.zeros((8, 118), F32)], axis=1)

    nf2 = _node_proj(node_feats, W_np, Wf1, b_np, g_np, be_np)
    gbt = _cond_proj(cond, Wc1, bc1, g_c, be_c, Wc2, bc2)
    S = _sc_gather_sum(nf2, src, dst, n_edges)
    return _edge_fuse(S, geo_t, abig, cvec, bounds, gbt)


# speedup vs baseline: 1.3676x; 1.3676x over previous
import functools

import jax
import jax.numpy as jnp
from jax import lax
from jax.experimental import pallas as pl
from jax.experimental.pallas import tpu as pltpu
from jax.experimental.pallas import tpu_sc as plsc

F32 = jnp.float32
EPS = 1e-5

NC, NS = 2, 16
NW = NC * NS


def _node_proj_body(x_ref, w_ref, wf1_ref, v_ref, o_ref):
    h = jnp.dot(x_ref[...], w_ref[...], preferred_element_type=F32)
    h = h + v_ref[0:1, :]
    m = jnp.mean(h, axis=-1, keepdims=True)
    hc = h - m
    var = jnp.mean(hc * hc, axis=-1, keepdims=True)
    y = hc * lax.rsqrt(var + EPS) * v_ref[1:2, :] + v_ref[2:3, :]
    o_ref[...] = jnp.dot(y, wf1_ref[...], preferred_element_type=F32)


def _node_proj(node_feats, W_np, Wf1, b_np, g_np, be_np):
    n, k = node_feats.shape
    blk = 2000
    grid = n // blk
    vecs = jnp.concatenate(
        [b_np[None], g_np[None], be_np[None], jnp.zeros((5, 128), F32)], axis=0
    )
    return pl.pallas_call(
        _node_proj_body,
        grid=(grid,),
        in_specs=[
            pl.BlockSpec((blk, k), lambda i: (i, 0)),
            pl.BlockSpec((k, 128), lambda i: (0, 0)),
            pl.BlockSpec((128, 128), lambda i: (0, 0)),
            pl.BlockSpec((8, 128), lambda i: (0, 0)),
        ],
        out_specs=pl.BlockSpec((blk, 128), lambda i: (i, 0)),
        out_shape=jax.ShapeDtypeStruct((n, 128), F32),
    )(node_feats, W_np, Wf1, vecs)


def _cond_proj_body(c_ref, w1_ref, w2_ref, v_ref, b2_ref, o_ref):
    h = jnp.dot(c_ref[...], w1_ref[...], preferred_element_type=F32)
    h = h + v_ref[0:1, :]
    m = jnp.mean(h, axis=-1, keepdims=True)
    hc = h - m
    var = jnp.mean(hc * hc, axis=-1, keepdims=True)
    y = hc * lax.rsqrt(var + EPS) * v_ref[1:2, :] + v_ref[2:3, :]
    gb = jnp.dot(y, w2_ref[...], preferred_element_type=F32) + b2_ref[0:1, :]
    lane = lax.broadcasted_iota(jnp.int32, gb.shape, 1)
    o_ref[...] = gb + (lane < 128).astype(F32)


def _cond_proj(cond, Wc1, bc1, g_c, be_c, Wc2, bc2):
    vecs = jnp.concatenate(
        [bc1[None], g_c[None], be_c[None], jnp.zeros((5, 128), F32)], axis=0
    )
    b2 = jnp.concatenate([bc2[None], jnp.zeros((7, 256), F32)], axis=0)
    return pl.pallas_call(
        _cond_proj_body,
        grid=(1,),
        in_specs=[
            pl.BlockSpec((16, 128), lambda i: (0, 0)),
            pl.BlockSpec((128, 128), lambda i: (0, 0)),
            pl.BlockSpec((128, 256), lambda i: (0, 0)),
            pl.BlockSpec((8, 128), lambda i: (0, 0)),
            pl.BlockSpec((8, 256), lambda i: (0, 0)),
        ],
        out_specs=pl.BlockSpec((16, 256), lambda i: (0, 0)),
        out_shape=jax.ShapeDtypeStruct((16, 256), F32),
    )(cond, Wc1, Wc2, vecs, b2)


def _sc_gather_sum(nf2, src, dst, n_edges):
    C = 128
    n_groups = n_edges // C
    base_g, extra = divmod(n_groups, NW)

    mesh = plsc.VectorSubcoreMesh(
        core_axis_name="c", subcore_axis_name="s", num_cores=NC, num_subcores=NS
    )

    @functools.partial(
        pl.kernel,
        out_type=jax.ShapeDtypeStruct((n_edges, 128), F32),
        mesh=mesh,
        scratch_types=[
            pltpu.VMEM((C,), jnp.int32),
            pltpu.VMEM((C,), jnp.int32),
            pltpu.VMEM((C, 128), F32),
            pltpu.VMEM((C, 128), F32),
            pltpu.SemaphoreType.DMA,
            pltpu.SemaphoreType.DMA,
        ],
    )
    def sc_kernel(nf2_hbm, src_hbm, dst_hbm, out_hbm, idx_s, idx_d, rows_a,
                  rows_b, sem_a, sem_b):
        wid = lax.axis_index("s") * NC + lax.axis_index("c")
        g0 = wid * base_g + jnp.minimum(wid, extra)
        ng = base_g + jnp.where(wid < extra, 1, 0)

        def group_body(i, carry):
            base = (g0 + i) * C
            pltpu.sync_copy(src_hbm.at[pl.ds(base, C)], idx_s)
            pltpu.sync_copy(dst_hbm.at[pl.ds(base, C)], idx_d)
            cp_a = pltpu.async_copy(nf2_hbm.at[idx_s], rows_a, sem_a)
            cp_b = pltpu.async_copy(nf2_hbm.at[idx_d], rows_b, sem_b)
            cp_a.wait()
            cp_b.wait()

            def add_body(r, c2):
                for cc in range(8):
                    sl = pl.ds(cc * 16, 16)
                    rows_a[r, sl] = rows_a[r, sl] + rows_b[r, sl]
                return c2

            lax.fori_loop(0, C, add_body, 0)
            pltpu.sync_copy(rows_a, out_hbm.at[pl.ds(base, C)])
            return carry

        lax.fori_loop(0, ng, group_body, 0)

    return sc_kernel(nf2, src, dst)


def _edge_body(s_ref, xt_ref, ab_ref, cv_ref, bnd_ref, gbt_ref, o_ref, *,
               blk):
    xt = xt_ref[...]
    s_in = s_ref[...]
    c1 = cv_ref[0:1, :]
    vrow = cv_ref[1:2, :]
    crow = cv_ref[2:3, :]
    bbar = cv_ref[3, 0]
    ccst = cv_ref[3, 1]

    big = lax.dot_general(
        xt, ab_ref[...], ((( 0,), (0,)), ((), ())),
        preferred_element_type=F32)
    t = big[:, :128] + c1
    mu = big[:, 128:129] + bbar
    q = big[:, 129:137]
    xu = big[:, 137:138]
    ones8 = jnp.full((8, 1), 1.0, F32)
    varg = (jnp.dot(q * q, ones8, preferred_element_type=F32)
            + 2.0 * xu + ccst)
    inv_sg = lax.rsqrt(varg + EPS)

    y0 = s_in + (t - mu * vrow) * inv_sg + crow
    ones128 = jnp.full((128, 1), 1.0 / 128.0, F32)
    m = jnp.dot(y0, ones128, preferred_element_type=F32)
    yc = y0 - m
    var = jnp.dot(yc * yc, ones128, preferred_element_type=F32)
    y = yc * lax.rsqrt(var + EPS)

    i = pl.program_id(0)
    gidx = i * blk + lax.broadcasted_iota(jnp.int32, (blk, 1), 0)
    starts = bnd_ref[0:1, :]
    ends = bnd_ref[1:2, :]
    oh = jnp.logical_and(gidx >= starts, gidx < ends).astype(F32)
    gb = jnp.dot(oh, gbt_ref[...], preferred_element_type=F32)
    o_ref[...] = jnp.maximum(y * gb[:, :128] + gb[:, 128:], 0.0)


def _edge_fuse(S, geo_t, abig, cvec, bounds, gbt):
    n_edges = S.shape[0]
    blk = 1280
    grid = n_edges // blk
    return pl.pallas_call(
        functools.partial(_edge_body, blk=blk),
        grid=(grid,),
        in_specs=[
            pl.BlockSpec((blk, 128), lambda i: (i, 0)),
            pl.BlockSpec((8, blk), lambda i: (0, i)),
            pl.BlockSpec((8, 256), lambda i: (0, 0)),
            pl.BlockSpec((8, 128), lambda i: (0, 0)),
            pl.BlockSpec((8, 16), lambda i: (0, 0)),
            pl.BlockSpec((16, 256), lambda i: (0, 0)),
        ],
        out_specs=pl.BlockSpec((blk, 128), lambda i: (i, 0)),
        out_shape=jax.ShapeDtypeStruct((n_edges, 128), F32),
    )(S, geo_t, abig, cvec, bounds, gbt)


def kernel(node_feats, edge_index, edge_geo, cond, batch_ids,
           W_np, b_np, g_np, be_np,
           W_geo, b_geo, g_geo, be_geo,
           Wc1, bc1, g_c, be_c, Wc2, bc2,
           Wf, bf):
    n_edges = edge_index.shape[1]
    src = edge_index[0].astype(jnp.int32)
    dst = edge_index[1].astype(jnp.int32)
    geo_t = edge_geo.T
    bids = batch_ids.astype(jnp.int32)
    starts = jnp.searchsorted(bids, jnp.arange(16, dtype=jnp.int32)).astype(jnp.int32)
    ends = jnp.concatenate([starts[1:], jnp.array([n_edges], jnp.int32)])
    bounds = jnp.concatenate(
        [starts[None], ends[None], jnp.zeros((6, 16), jnp.int32)], axis=0)

    Wf1 = Wf[:128]
    Wf2 = Wf[128:]

    Wgsum = W_geo.reshape(8, 8, 128).sum(axis=0)
    A = Wgsum @ (g_geo[:, None] * Wf2)
    c1 = (b_geo * g_geo) @ Wf2
    vrow = g_geo @ Wf2
    crow = be_geo @ Wf2 + bf
    m8 = Wgsum.mean(axis=1)
    bbar = b_geo.mean()
    acen = Wgsum - m8[:, None]
    bcen = b_geo - bbar
    Mc = (acen @ acen.T) / 128.0
    uc = (acen @ bcen) / 128.0
    ccst = jnp.dot(bcen, bcen) / 128.0
    L = jnp.linalg.cholesky(Mc + 1e-12 * jnp.eye(8, dtype=F32))

    misc = jnp.zeros((128,), F32).at[0].set(bbar).at[1].set(ccst)
    cvec = jnp.concatenate(
        [c1[None], vrow[None], crow[None], misc[None], jnp.zeros((4, 128), F32)],
        axis=0,
    )
    abig = jnp.concatenate(
        [A, m8[:, None], L, uc[:, None], jnp.zeros((8, 118), F32)], axis=1)

    nf2 = _node_proj(node_feats, W_np, Wf1, b_np, g_np, be_np)
    gbt = _cond_proj(cond, Wc1, bc1, g_c, be_c, Wc2, bc2)
    S = _sc_gather_sum(nf2, src, dst, n_edges)
    return _edge_fuse(S, geo_t, abig, cvec, bounds, gbt)


# speedup vs baseline: 1.7782x; 1.3002x over previous
import functools

import jax
import jax.numpy as jnp
from jax import lax
from jax.experimental import pallas as pl
from jax.experimental.pallas import tpu as pltpu
from jax.experimental.pallas import tpu_sc as plsc

F32 = jnp.float32
EPS = 1e-5

NC, NS = 2, 16
NW = NC * NS


def _node_proj_body(x_ref, w_ref, wf1_ref, v_ref, o_ref):
    h = jnp.dot(x_ref[...], w_ref[...], preferred_element_type=F32)
    h = h + v_ref[0:1, :]
    m = jnp.mean(h, axis=-1, keepdims=True)
    hc = h - m
    var = jnp.mean(hc * hc, axis=-1, keepdims=True)
    y = hc * lax.rsqrt(var + EPS) * v_ref[1:2, :] + v_ref[2:3, :]
    o_ref[...] = jnp.dot(y, wf1_ref[...], preferred_element_type=F32)


def _node_proj(node_feats, W_np, Wf1, b_np, g_np, be_np):
    n, k = node_feats.shape
    blk = 2000
    grid = n // blk
    vecs = jnp.concatenate(
        [b_np[None], g_np[None], be_np[None], jnp.zeros((5, 128), F32)], axis=0
    )
    return pl.pallas_call(
        _node_proj_body,
        grid=(grid,),
        in_specs=[
            pl.BlockSpec((blk, k), lambda i: (i, 0)),
            pl.BlockSpec((k, 128), lambda i: (0, 0)),
            pl.BlockSpec((128, 128), lambda i: (0, 0)),
            pl.BlockSpec((8, 128), lambda i: (0, 0)),
        ],
        out_specs=pl.BlockSpec((blk, 128), lambda i: (i, 0)),
        out_shape=jax.ShapeDtypeStruct((n, 128), F32),
    )(node_feats, W_np, Wf1, vecs)


def _cond_proj_body(c_ref, w1_ref, w2_ref, v_ref, b2_ref, o_ref):
    h = jnp.dot(c_ref[...], w1_ref[...], preferred_element_type=F32)
    h = h + v_ref[0:1, :]
    m = jnp.mean(h, axis=-1, keepdims=True)
    hc = h - m
    var = jnp.mean(hc * hc, axis=-1, keepdims=True)
    y = hc * lax.rsqrt(var + EPS) * v_ref[1:2, :] + v_ref[2:3, :]
    gb = jnp.dot(y, w2_ref[...], preferred_element_type=F32) + b2_ref[0:1, :]
    lane = lax.broadcasted_iota(jnp.int32, gb.shape, 1)
    o_ref[...] = gb + (lane < 128).astype(F32)


def _cond_proj(cond, Wc1, bc1, g_c, be_c, Wc2, bc2):
    vecs = jnp.concatenate(
        [bc1[None], g_c[None], be_c[None], jnp.zeros((5, 128), F32)], axis=0
    )
    b2 = jnp.concatenate([bc2[None], jnp.zeros((7, 256), F32)], axis=0)
    return pl.pallas_call(
        _cond_proj_body,
        grid=(1,),
        in_specs=[
            pl.BlockSpec((16, 128), lambda i: (0, 0)),
            pl.BlockSpec((128, 128), lambda i: (0, 0)),
            pl.BlockSpec((128, 256), lambda i: (0, 0)),
            pl.BlockSpec((8, 128), lambda i: (0, 0)),
            pl.BlockSpec((8, 256), lambda i: (0, 0)),
        ],
        out_specs=pl.BlockSpec((16, 256), lambda i: (0, 0)),
        out_shape=jax.ShapeDtypeStruct((16, 256), F32),
    )(cond, Wc1, Wc2, vecs, b2)


def _sc_gather_sum(nf2, src, dst, n_edges):
    C = 128
    n_groups = n_edges // C
    base_g = n_groups // NW
    extra = n_groups - base_g * NW
    per_w = base_g * C
    half = base_g // 2

    mesh = plsc.VectorSubcoreMesh(
        core_axis_name="c", subcore_axis_name="s", num_cores=NC, num_subcores=NS
    )

    @functools.partial(
        pl.kernel,
        out_type=jax.ShapeDtypeStruct((n_edges, 128), F32),
        mesh=mesh,
        scratch_types=[
            pltpu.VMEM((per_w,), jnp.int32),
            pltpu.VMEM((per_w,), jnp.int32),
            pltpu.VMEM((C, 128), F32),
            pltpu.VMEM((C, 128), F32),
            pltpu.VMEM((C, 128), F32),
            pltpu.VMEM((C, 128), F32),
            pltpu.VMEM((C,), jnp.int32),
            pltpu.VMEM((C,), jnp.int32),
            pltpu.SemaphoreType.DMA,
            pltpu.SemaphoreType.DMA,
            pltpu.SemaphoreType.DMA,
            pltpu.SemaphoreType.DMA,
            pltpu.SemaphoreType.DMA,
            pltpu.SemaphoreType.DMA,
        ],
    )
    def sc_kernel(nf2_hbm, src_hbm, dst_hbm, out_hbm,
                  ixs, ixd, ra0, rb0, ra1, rb1, xis, xid,
                  sga0, sgb0, sga1, sgb1, swb0, swb1):
        wid = lax.axis_index("s") * NC + lax.axis_index("c")
        e0 = wid * per_w
        pltpu.sync_copy(src_hbm.at[pl.ds(e0, per_w)], ixs)
        pltpu.sync_copy(dst_hbm.at[pl.ds(e0, per_w)], ixd)

        def start_gathers(i, ra, rb, sga, sgb):
            sl = pl.ds(i * C, C)
            pltpu.async_copy(nf2_hbm.at[ixs.at[sl]], ra, sga)
            pltpu.async_copy(nf2_hbm.at[ixd.at[sl]], rb, sgb)

        def wait_gathers(ra, rb, sga, sgb):
            pltpu.make_async_copy(nf2_hbm.at[ixs.at[pl.ds(0, C)]], ra, sga).wait()
            pltpu.make_async_copy(nf2_hbm.at[ixd.at[pl.ds(0, C)]], rb, sgb).wait()

        def add_rows(ra, rb):
            def add_body(r, c2):
                for cc in range(8):
                    sl = pl.ds(cc * 16, 16)
                    ra[r, sl] = ra[r, sl] + rb[r, sl]
                return c2
            lax.fori_loop(0, C, add_body, 0)

        def start_wb(i, ra, swb):
            pltpu.async_copy(ra, out_hbm.at[pl.ds(e0 + i * C, C)], swb)

        def wait_wb(ra, swb):
            pltpu.make_async_copy(ra, out_hbm.at[pl.ds(e0, C)], swb).wait()

        start_gathers(0, ra0, rb0, sga0, sgb0)

        def body(j, carry):
            @pl.when(j > 0)
            def _():
                wait_wb(ra1, swb1)
            start_gathers(2 * j + 1, ra1, rb1, sga1, sgb1)
            wait_gathers(ra0, rb0, sga0, sgb0)
            add_rows(ra0, rb0)
            start_wb(2 * j, ra0, swb0)

            @pl.when(j < half - 1)
            def _():
                wait_wb(ra0, swb0)
                start_gathers(2 * j + 2, ra0, rb0, sga0, sgb0)

            wait_gathers(ra1, rb1, sga1, sgb1)
            add_rows(ra1, rb1)
            start_wb(2 * j + 1, ra1, swb1)
            return carry

        lax.fori_loop(0, half, body, 0)
        wait_wb(ra0, swb0)
        wait_wb(ra1, swb1)

        @pl.when(wid < extra)
        def _():
            base = (NW * base_g + wid) * C
            pltpu.sync_copy(src_hbm.at[pl.ds(base, C)], xis)
            pltpu.sync_copy(dst_hbm.at[pl.ds(base, C)], xid)
            cpa = pltpu.async_copy(nf2_hbm.at[xis], ra0, sga0)
            cpb = pltpu.async_copy(nf2_hbm.at[xid], rb0, sgb0)
            cpa.wait()
            cpb.wait()
            add_rows(ra0, rb0)
            pltpu.sync_copy(ra0, out_hbm.at[pl.ds(base, C)])

    return sc_kernel(nf2, src, dst)


def _edge_body(s_ref, xt_ref, ab_ref, cv_ref, bnd_ref, gbt_ref, o_ref, *,
               blk):
    xt = xt_ref[...]
    s_in = s_ref[...]
    c1 = cv_ref[0:1, :]
    vrow = cv_ref[1:2, :]
    crow = cv_ref[2:3, :]
    bbar = cv_ref[3, 0]
    ccst = cv_ref[3, 1]

    big = lax.dot_general(
        xt, ab_ref[...], ((( 0,), (0,)), ((), ())),
        preferred_element_type=F32)
    t = big[:, :128] + c1
    mu = big[:, 128:129] + bbar
    q = big[:, 129:137]
    xu = big[:, 137:138]
    ones8 = jnp.full((8, 1), 1.0, F32)
    varg = (jnp.dot(q * q, ones8, preferred_element_type=F32)
            + 2.0 * xu + ccst)
    inv_sg = lax.rsqrt(varg + EPS)

    y0 = s_in + (t - mu * vrow) * inv_sg + crow
    ones128 = jnp.full((128, 1), 1.0 / 128.0, F32)
    m = jnp.dot(y0, ones128, preferred_element_type=F32)
    yc = y0 - m
    var = jnp.dot(yc * yc, ones128, preferred_element_type=F32)
    y = yc * lax.rsqrt(var + EPS)

    i = pl.program_id(0)
    gidx = i * blk + lax.broadcasted_iota(jnp.int32, (blk, 1), 0)
    starts = bnd_ref[0:1, :]
    ends = bnd_ref[1:2, :]
    oh = jnp.logical_and(gidx >= starts, gidx < ends).astype(F32)
    gb = jnp.dot(oh, gbt_ref[...], preferred_element_type=F32)
    o_ref[...] = jnp.maximum(y * gb[:, :128] + gb[:, 128:], 0.0)


def _edge_fuse(S, geo_t, abig, cvec, bounds, gbt):
    n_edges = S.shape[0]
    blk = 1280
    grid = n_edges // blk
    return pl.pallas_call(
        functools.partial(_edge_body, blk=blk),
        grid=(grid,),
        in_specs=[
            pl.BlockSpec((blk, 128), lambda i: (i, 0)),
            pl.BlockSpec((8, blk), lambda i: (0, i)),
            pl.BlockSpec((8, 256), lambda i: (0, 0)),
            pl.BlockSpec((8, 128), lambda i: (0, 0)),
            pl.BlockSpec((8, 16), lambda i: (0, 0)),
            pl.BlockSpec((16, 256), lambda i: (0, 0)),
        ],
        out_specs=pl.BlockSpec((blk, 128), lambda i: (i, 0)),
        out_shape=jax.ShapeDtypeStruct((n_edges, 128), F32),
    )(S, geo_t, abig, cvec, bounds, gbt)


def kernel(node_feats, edge_index, edge_geo, cond, batch_ids,
           W_np, b_np, g_np, be_np,
           W_geo, b_geo, g_geo, be_geo,
           Wc1, bc1, g_c, be_c, Wc2, bc2,
           Wf, bf):
    n_edges = edge_index.shape[1]
    src = edge_index[0].astype(jnp.int32)
    dst = edge_index[1].astype(jnp.int32)
    geo_t = edge_geo.T
    bids = batch_ids.astype(jnp.int32)
    starts = jnp.searchsorted(bids, jnp.arange(16, dtype=jnp.int32)).astype(jnp.int32)
    ends = jnp.concatenate([starts[1:], jnp.array([n_edges], jnp.int32)])
    bounds = jnp.concatenate(
        [starts[None], ends[None], jnp.zeros((6, 16), jnp.int32)], axis=0)

    Wf1 = Wf[:128]
    Wf2 = Wf[128:]

    Wgsum = W_geo.reshape(8, 8, 128).sum(axis=0)
    A = Wgsum @ (g_geo[:, None] * Wf2)
    c1 = (b_geo * g_geo) @ Wf2
    vrow = g_geo @ Wf2
    crow = be_geo @ Wf2 + bf
    m8 = Wgsum.mean(axis=1)
    bbar = b_geo.mean()
    acen = Wgsum - m8[:, None]
    bcen = b_geo - bbar
    Mc = (acen @ acen.T) / 128.0
    uc = (acen @ bcen) / 128.0
    ccst = jnp.dot(bcen, bcen) / 128.0
    L = jnp.linalg.cholesky(Mc + 1e-12 * jnp.eye(8, dtype=F32))

    misc = jnp.zeros((128,), F32).at[0].set(bbar).at[1].set(ccst)
    cvec = jnp.concatenate(
        [c1[None], vrow[None], crow[None], misc[None], jnp.zeros((4, 128), F32)],
        axis=0,
    )
    abig = jnp.concatenate(
        [A, m8[:, None], L, uc[:, None], jnp.zeros((8, 118), F32)], axis=1)

    nf2 = _node_proj(node_feats, W_np, Wf1, b_np, g_np, be_np)
    gbt = _cond_proj(cond, Wc1, bc1, g_c, be_c, Wc2, bc2)
    S = _sc_gather_sum(nf2, src, dst, n_edges)
    return _edge_fuse(S, geo_t, abig, cvec, bounds, gbt)


# speedup vs baseline: 2.3937x; 1.3461x over previous
import functools

import jax
import jax.numpy as jnp
from jax import lax
from jax.experimental import pallas as pl
from jax.experimental.pallas import tpu as pltpu
from jax.experimental.pallas import tpu_sc as plsc

F32 = jnp.float32
EPS = 1e-5

NC, NS = 2, 16
NW = NC * NS


def _node_proj_body(x_ref, w_ref, wf1_ref, v_ref, o_ref):
    h = jnp.dot(x_ref[...], w_ref[...], preferred_element_type=F32)
    h = h + v_ref[0:1, :]
    m = jnp.mean(h, axis=-1, keepdims=True)
    hc = h - m
    var = jnp.mean(hc * hc, axis=-1, keepdims=True)
    y = hc * lax.rsqrt(var + EPS) * v_ref[1:2, :] + v_ref[2:3, :]
    o_ref[...] = jnp.dot(y, wf1_ref[...], preferred_element_type=F32)


def _node_proj(node_feats, W_np, Wf1, b_np, g_np, be_np):
    n, k = node_feats.shape
    blk = 2000
    grid = n // blk
    vecs = jnp.concatenate(
        [b_np[None], g_np[None], be_np[None], jnp.zeros((5, 128), F32)], axis=0
    )
    return pl.pallas_call(
        _node_proj_body,
        grid=(grid,),
        in_specs=[
            pl.BlockSpec((blk, k), lambda i: (i, 0)),
            pl.BlockSpec((k, 128), lambda i: (0, 0)),
            pl.BlockSpec((128, 128), lambda i: (0, 0)),
            pl.BlockSpec((8, 128), lambda i: (0, 0)),
        ],
        out_specs=pl.BlockSpec((blk, 128), lambda i: (i, 0)),
        out_shape=jax.ShapeDtypeStruct((n, 128), F32),
    )(node_feats, W_np, Wf1, vecs)


def _cond_proj_body(c_ref, w1_ref, w2_ref, v_ref, b2_ref, o_ref):
    h = jnp.dot(c_ref[...], w1_ref[...], preferred_element_type=F32)
    h = h + v_ref[0:1, :]
    m = jnp.mean(h, axis=-1, keepdims=True)
    hc = h - m
    var = jnp.mean(hc * hc, axis=-1, keepdims=True)
    y = hc * lax.rsqrt(var + EPS) * v_ref[1:2, :] + v_ref[2:3, :]
    gb = jnp.dot(y, w2_ref[...], preferred_element_type=F32) + b2_ref[0:1, :]
    lane = lax.broadcasted_iota(jnp.int32, gb.shape, 1)
    o_ref[...] = gb + (lane < 128).astype(F32)


def _cond_proj(cond, Wc1, bc1, g_c, be_c, Wc2, bc2):
    vecs = jnp.concatenate(
        [bc1[None], g_c[None], be_c[None], jnp.zeros((5, 128), F32)], axis=0
    )
    b2 = jnp.concatenate([bc2[None], jnp.zeros((7, 256), F32)], axis=0)
    return pl.pallas_call(
        _cond_proj_body,
        grid=(1,),
        in_specs=[
            pl.BlockSpec((16, 128), lambda i: (0, 0)),
            pl.BlockSpec((128, 128), lambda i: (0, 0)),
            pl.BlockSpec((128, 256), lambda i: (0, 0)),
            pl.BlockSpec((8, 128), lambda i: (0, 0)),
            pl.BlockSpec((8, 256), lambda i: (0, 0)),
        ],
        out_specs=pl.BlockSpec((16, 256), lambda i: (0, 0)),
        out_shape=jax.ShapeDtypeStruct((16, 256), F32),
    )(cond, Wc1, Wc2, vecs, b2)


def _sc_gather_sum(nf2, src, dst, n_edges):
    C = 128
    n_groups = n_edges // C
    base_g = n_groups // NW
    extra = n_groups - base_g * NW
    per_w = base_g * C
    half = base_g // 2
    odd = base_g - 2 * half

    mesh = plsc.VectorSubcoreMesh(
        core_axis_name="c", subcore_axis_name="s", num_cores=NC, num_subcores=NS
    )

    @functools.partial(
        pl.kernel,
        out_type=jax.ShapeDtypeStruct((n_edges, 128), F32),
        mesh=mesh,
        scratch_types=[
            pltpu.VMEM((per_w,), jnp.int32),
            pltpu.VMEM((per_w,), jnp.int32),
            pltpu.VMEM((C, 128), F32),
            pltpu.VMEM((C, 128), F32),
            pltpu.VMEM((C, 128), F32),
            pltpu.VMEM((C, 128), F32),
            pltpu.VMEM((C,), jnp.int32),
            pltpu.VMEM((C,), jnp.int32),
            pltpu.SemaphoreType.DMA,
            pltpu.SemaphoreType.DMA,
            pltpu.SemaphoreType.DMA,
            pltpu.SemaphoreType.DMA,
            pltpu.SemaphoreType.DMA,
            pltpu.SemaphoreType.DMA,
        ],
    )
    def sc_kernel(nf2_hbm, src_hbm, dst_hbm, out_hbm,
                  ixs, ixd, ra0, rb0, ra1, rb1, xis, xid,
                  sga0, sgb0, sga1, sgb1, swb0, swb1):
        wid = lax.axis_index("s") * NC + lax.axis_index("c")
        e0 = wid * per_w
        pltpu.sync_copy(src_hbm.at[pl.ds(e0, per_w)], ixs)
        pltpu.sync_copy(dst_hbm.at[pl.ds(e0, per_w)], ixd)

        def start_gathers(i, ra, rb, sga, sgb):
            sl = pl.ds(i * C, C)
            pltpu.async_copy(nf2_hbm.at[ixs.at[sl]], ra, sga)
            pltpu.async_copy(nf2_hbm.at[ixd.at[sl]], rb, sgb)

        def wait_gathers(ra, rb, sga, sgb):
            pltpu.make_async_copy(nf2_hbm.at[ixs.at[pl.ds(0, C)]], ra, sga).wait()
            pltpu.make_async_copy(nf2_hbm.at[ixd.at[pl.ds(0, C)]], rb, sgb).wait()

        def add_rows(ra, rb):
            def add_body(r, c2):
                for cc in range(8):
                    sl = pl.ds(cc * 16, 16)
                    ra[r, sl] = ra[r, sl] + rb[r, sl]
                return c2
            lax.fori_loop(0, C, add_body, 0)

        def start_wb(i, ra, swb):
            pltpu.async_copy(ra, out_hbm.at[pl.ds(e0 + i * C, C)], swb)

        def wait_wb(ra, swb):
            pltpu.make_async_copy(ra, out_hbm.at[pl.ds(e0, C)], swb).wait()

        start_gathers(0, ra0, rb0, sga0, sgb0)

        def body(j, carry):
            @pl.when(j > 0)
            def _():
                wait_wb(ra1, swb1)
            start_gathers(2 * j + 1, ra1, rb1, sga1, sgb1)
            wait_gathers(ra0, rb0, sga0, sgb0)
            add_rows(ra0, rb0)
            start_wb(2 * j, ra0, swb0)

            @pl.when(j < half - 1)
            def _():
                wait_wb(ra0, swb0)
                start_gathers(2 * j + 2, ra0, rb0, sga0, sgb0)

            wait_gathers(ra1, rb1, sga1, sgb1)
            add_rows(ra1, rb1)
            start_wb(2 * j + 1, ra1, swb1)
            return carry

        if half == 0:
            wait_gathers(ra0, rb0, sga0, sgb0)
            add_rows(ra0, rb0)
            start_wb(0, ra0, swb0)
            wait_wb(ra0, swb0)
        else:
            lax.fori_loop(0, half, body, 0)
            if odd:
                wait_wb(ra0, swb0)
                start_gathers(base_g - 1, ra0, rb0, sga0, sgb0)
                wait_wb(ra1, swb1)
                wait_gathers(ra0, rb0, sga0, sgb0)
                add_rows(ra0, rb0)
                start_wb(base_g - 1, ra0, swb0)
                wait_wb(ra0, swb0)
            else:
                wait_wb(ra0, swb0)
                wait_wb(ra1, swb1)

        @pl.when(wid < extra)
        def _():
            base = (NW * base_g + wid) * C
            pltpu.sync_copy(src_hbm.at[pl.ds(base, C)], xis)
            pltpu.sync_copy(dst_hbm.at[pl.ds(base, C)], xid)
            cpa = pltpu.async_copy(nf2_hbm.at[xis], ra0, sga0)
            cpb = pltpu.async_copy(nf2_hbm.at[xid], rb0, sgb0)
            cpa.wait()
            cpb.wait()
            add_rows(ra0, rb0)
            pltpu.sync_copy(ra0, out_hbm.at[pl.ds(base, C)])

    return sc_kernel(nf2, src, dst)


def _edge_body(prev_ref, s_ref, xt_ref, ab_ref, cv_ref, bnd_ref, gbt_ref,
               o_ref, *, blk, blk_off):
    del prev_ref
    xt = xt_ref[...]
    s_in = s_ref[...]
    c1 = cv_ref[0:1, :]
    vrow = cv_ref[1:2, :]
    crow = cv_ref[2:3, :]
    bbar = cv_ref[3, 0]
    ccst = cv_ref[3, 1]

    big = lax.dot_general(
        xt, ab_ref[...], ((( 0,), (0,)), ((), ())),
        preferred_element_type=F32)
    t = big[:, :128] + c1
    mu = big[:, 128:129] + bbar
    q = big[:, 129:137]
    xu = big[:, 137:138]
    ones8 = jnp.full((8, 1), 1.0, F32)
    varg = (jnp.dot(q * q, ones8, preferred_element_type=F32)
            + 2.0 * xu + ccst)
    inv_sg = lax.rsqrt(varg + EPS)

    y0 = s_in + (t - mu * vrow) * inv_sg + crow
    ones128 = jnp.full((128, 1), 1.0 / 128.0, F32)
    m = jnp.dot(y0, ones128, preferred_element_type=F32)
    yc = y0 - m
    var = jnp.dot(yc * yc, ones128, preferred_element_type=F32)
    y = yc * lax.rsqrt(var + EPS)

    i = pl.program_id(0) + blk_off
    gidx = i * blk + lax.broadcasted_iota(jnp.int32, (blk, 1), 0)
    starts = bnd_ref[0:1, :]
    ends = bnd_ref[1:2, :]
    oh = jnp.logical_and(gidx >= starts, gidx < ends).astype(F32)
    gb = jnp.dot(oh, gbt_ref[...], preferred_element_type=F32)
    o_ref[...] = jnp.maximum(y * gb[:, :128] + gb[:, 128:], 0.0)


def _edge_fuse_chunk(prev, S, geo_t, abig, cvec, bounds, gbt, *, n_edges,
                     blk, blk_off, first):
    grid = S.shape[0] // blk
    return pl.pallas_call(
        functools.partial(_edge_body, blk=blk, blk_off=blk_off),
        grid=(grid,),
        in_specs=[
            pl.BlockSpec(memory_space=pl.ANY),
            pl.BlockSpec((blk, 128), lambda i: (i, 0)),
            pl.BlockSpec((8, blk), lambda i: (0, i + blk_off)),
            pl.BlockSpec((8, 256), lambda i: (0, 0)),
            pl.BlockSpec((8, 128), lambda i: (0, 0)),
            pl.BlockSpec((8, 16), lambda i: (0, 0)),
            pl.BlockSpec((16, 256), lambda i: (0, 0)),
        ],
        out_specs=pl.BlockSpec((blk, 128), lambda i: (i + blk_off, 0)),
        out_shape=jax.ShapeDtypeStruct((n_edges, 128), F32),
        input_output_aliases=({} if first else {0: 0}),
    )(prev, S, geo_t, abig, cvec, bounds, gbt)


def kernel(node_feats, edge_index, edge_geo, cond, batch_ids,
           W_np, b_np, g_np, be_np,
           W_geo, b_geo, g_geo, be_geo,
           Wc1, bc1, g_c, be_c, Wc2, bc2,
           Wf, bf):
    n_edges = edge_index.shape[1]
    src = edge_index[0].astype(jnp.int32)
    dst = edge_index[1].astype(jnp.int32)
    geo_t = edge_geo.T
    bids = batch_ids.astype(jnp.int32)
    starts = jnp.searchsorted(bids, jnp.arange(16, dtype=jnp.int32)).astype(jnp.int32)
    ends = jnp.concatenate([starts[1:], jnp.array([n_edges], jnp.int32)])
    bounds = jnp.concatenate(
        [starts[None], ends[None], jnp.zeros((6, 16), jnp.int32)], axis=0)

    Wf1 = Wf[:128]
    Wf2 = Wf[128:]

    Wgsum = W_geo.reshape(8, 8, 128).sum(axis=0)
    A = Wgsum @ (g_geo[:, None] * Wf2)
    c1 = (b_geo * g_geo) @ Wf2
    vrow = g_geo @ Wf2
    crow = be_geo @ Wf2 + bf
    m8 = Wgsum.mean(axis=1)
    bbar = b_geo.mean()
    acen = Wgsum - m8[:, None]
    bcen = b_geo - bbar
    Mc = (acen @ acen.T) / 128.0
    uc = (acen @ bcen) / 128.0
    ccst = jnp.dot(bcen, bcen) / 128.0
    L = jnp.linalg.cholesky(Mc + 1e-12 * jnp.eye(8, dtype=F32))

    misc = jnp.zeros((128,), F32).at[0].set(bbar).at[1].set(ccst)
    cvec = jnp.concatenate(
        [c1[None], vrow[None], crow[None], misc[None], jnp.zeros((4, 128), F32)],
        axis=0,
    )
    abig = jnp.concatenate(
        [A, m8[:, None], L, uc[:, None], jnp.zeros((8, 118), F32)], axis=1)

    nf2 = _node_proj(node_feats, W_np, Wf1, b_np, g_np, be_np)
    gbt = _cond_proj(cond, Wc1, bc1, g_c, be_c, Wc2, bc2)

    nch = 4
    blk = 3200
    ch = n_edges // nch
    out = jnp.zeros((8, 128), F32)
    for k in range(nch):
        s_k = _sc_gather_sum(nf2, src[k * ch:(k + 1) * ch],
                             dst[k * ch:(k + 1) * ch], ch)
        out = _edge_fuse_chunk(out, s_k, geo_t, abig, cvec, bounds, gbt,
                               n_edges=n_edges, blk=blk,
                               blk_off=k * (ch // blk), first=(k == 0))
    return out


# speedup vs baseline: 2.8677x; 1.1980x over previous
import functools

import jax
import jax.numpy as jnp
from jax import lax
from jax.experimental import pallas as pl
from jax.experimental.pallas import tpu as pltpu
from jax.experimental.pallas import tpu_sc as plsc

F32 = jnp.float32
EPS = 1e-5

NC, NS = 2, 16
NW = NC * NS


def _node_proj_body(x_ref, w_ref, wf1_ref, v_ref, o_ref):
    h = jnp.dot(x_ref[...], w_ref[...], preferred_element_type=F32)
    h = h + v_ref[0:1, :]
    m = jnp.mean(h, axis=-1, keepdims=True)
    hc = h - m
    var = jnp.mean(hc * hc, axis=-1, keepdims=True)
    y = hc * lax.rsqrt(var + EPS) * v_ref[1:2, :] + v_ref[2:3, :]
    o_ref[...] = jnp.dot(y, wf1_ref[...], preferred_element_type=F32)


def _node_proj(node_feats, W_np, Wf1, b_np, g_np, be_np):
    n, k = node_feats.shape
    blk = 2000
    grid = n // blk
    vecs = jnp.concatenate(
        [b_np[None], g_np[None], be_np[None], jnp.zeros((5, 128), F32)], axis=0
    )
    return pl.pallas_call(
        _node_proj_body,
        grid=(grid,),
        in_specs=[
            pl.BlockSpec((blk, k), lambda i: (i, 0)),
            pl.BlockSpec((k, 128), lambda i: (0, 0)),
            pl.BlockSpec((128, 128), lambda i: (0, 0)),
            pl.BlockSpec((8, 128), lambda i: (0, 0)),
        ],
        out_specs=pl.BlockSpec((blk, 128), lambda i: (i, 0)),
        out_shape=jax.ShapeDtypeStruct((n, 128), F32),
    )(node_feats, W_np, Wf1, vecs)


def _cond_proj_body(c_ref, w1_ref, w2_ref, v_ref, b2_ref, o_ref):
    h = jnp.dot(c_ref[...], w1_ref[...], preferred_element_type=F32)
    h = h + v_ref[0:1, :]
    m = jnp.mean(h, axis=-1, keepdims=True)
    hc = h - m
    var = jnp.mean(hc * hc, axis=-1, keepdims=True)
    y = hc * lax.rsqrt(var + EPS) * v_ref[1:2, :] + v_ref[2:3, :]
    gb = jnp.dot(y, w2_ref[...], preferred_element_type=F32) + b2_ref[0:1, :]
    lane = lax.broadcasted_iota(jnp.int32, gb.shape, 1)
    o_ref[...] = gb + (lane < 128).astype(F32)


def _cond_proj(cond, Wc1, bc1, g_c, be_c, Wc2, bc2):
    vecs = jnp.concatenate(
        [bc1[None], g_c[None], be_c[None], jnp.zeros((5, 128), F32)], axis=0
    )
    b2 = jnp.concatenate([bc2[None], jnp.zeros((7, 256), F32)], axis=0)
    return pl.pallas_call(
        _cond_proj_body,
        grid=(1,),
        in_specs=[
            pl.BlockSpec((16, 128), lambda i: (0, 0)),
            pl.BlockSpec((128, 128), lambda i: (0, 0)),
            pl.BlockSpec((128, 256), lambda i: (0, 0)),
            pl.BlockSpec((8, 128), lambda i: (0, 0)),
            pl.BlockSpec((8, 256), lambda i: (0, 0)),
        ],
        out_specs=pl.BlockSpec((16, 256), lambda i: (0, 0)),
        out_shape=jax.ShapeDtypeStruct((16, 256), F32),
    )(cond, Wc1, Wc2, vecs, b2)


def _sc_gather_sum(nf2, src, dst, n_edges):
    C = 128
    n_groups = n_edges // C
    base_g = n_groups // NW
    extra = n_groups - base_g * NW
    per_w = base_g * C
    half = base_g // 2
    odd = base_g - 2 * half

    mesh = plsc.VectorSubcoreMesh(
        core_axis_name="c", subcore_axis_name="s", num_cores=NC, num_subcores=NS
    )

    @functools.partial(
        pl.kernel,
        out_type=jax.ShapeDtypeStruct((n_edges, 128), F32),
        mesh=mesh,
        scratch_types=[
            pltpu.VMEM((per_w,), jnp.int32),
            pltpu.VMEM((per_w,), jnp.int32),
            pltpu.VMEM((C, 128), F32),
            pltpu.VMEM((C, 128), F32),
            pltpu.VMEM((C, 128), F32),
            pltpu.VMEM((C, 128), F32),
            pltpu.VMEM((C,), jnp.int32),
            pltpu.VMEM((C,), jnp.int32),
            pltpu.SemaphoreType.DMA,
            pltpu.SemaphoreType.DMA,
            pltpu.SemaphoreType.DMA,
            pltpu.SemaphoreType.DMA,
            pltpu.SemaphoreType.DMA,
            pltpu.SemaphoreType.DMA,
        ],
    )
    def sc_kernel(nf2_hbm, src_hbm, dst_hbm, out_hbm,
                  ixs, ixd, ra0, rb0, ra1, rb1, xis, xid,
                  sga0, sgb0, sga1, sgb1, swb0, swb1):
        wid = lax.axis_index("s") * NC + lax.axis_index("c")
        e0 = wid * per_w
        pltpu.sync_copy(src_hbm.at[pl.ds(e0, per_w)], ixs)
        pltpu.sync_copy(dst_hbm.at[pl.ds(e0, per_w)], ixd)

        def start_gathers(i, ra, rb, sga, sgb):
            sl = pl.ds(i * C, C)
            pltpu.async_copy(nf2_hbm.at[ixs.at[sl]], ra, sga)
            pltpu.async_copy(nf2_hbm.at[ixd.at[sl]], rb, sgb)

        def wait_gathers(ra, rb, sga, sgb):
            pltpu.make_async_copy(nf2_hbm.at[ixs.at[pl.ds(0, C)]], ra, sga).wait()
            pltpu.make_async_copy(nf2_hbm.at[ixd.at[pl.ds(0, C)]], rb, sgb).wait()

        def add_rows(ra, rb):
            def add_body(r, c2):
                for cc in range(8):
                    sl = pl.ds(cc * 16, 16)
                    ra[r, sl] = ra[r, sl] + rb[r, sl]
                return c2
            lax.fori_loop(0, C, add_body, 0)

        def start_wb(i, ra, swb):
            pltpu.async_copy(ra, out_hbm.at[pl.ds(e0 + i * C, C)], swb)

        def wait_wb(ra, swb):
            pltpu.make_async_copy(ra, out_hbm.at[pl.ds(e0, C)], swb).wait()

        start_gathers(0, ra0, rb0, sga0, sgb0)

        def body(j, carry):
            @pl.when(j > 0)
            def _():
                wait_wb(ra1, swb1)
            start_gathers(2 * j + 1, ra1, rb1, sga1, sgb1)
            wait_gathers(ra0, rb0, sga0, sgb0)
            add_rows(ra0, rb0)
            start_wb(2 * j, ra0, swb0)

            @pl.when(j < half - 1)
            def _():
                wait_wb(ra0, swb0)
                start_gathers(2 * j + 2, ra0, rb0, sga0, sgb0)

            wait_gathers(ra1, rb1, sga1, sgb1)
            add_rows(ra1, rb1)
            start_wb(2 * j + 1, ra1, swb1)
            return carry

        if half == 0:
            wait_gathers(ra0, rb0, sga0, sgb0)
            add_rows(ra0, rb0)
            start_wb(0, ra0, swb0)
            wait_wb(ra0, swb0)
        else:
            lax.fori_loop(0, half, body, 0)
            if odd:
                wait_wb(ra0, swb0)
                start_gathers(base_g - 1, ra0, rb0, sga0, sgb0)
                wait_wb(ra1, swb1)
                wait_gathers(ra0, rb0, sga0, sgb0)
                add_rows(ra0, rb0)
                start_wb(base_g - 1, ra0, swb0)
                wait_wb(ra0, swb0)
            else:
                wait_wb(ra0, swb0)
                wait_wb(ra1, swb1)

        @pl.when(wid < extra)
        def _():
            base = (NW * base_g + wid) * C
            pltpu.sync_copy(src_hbm.at[pl.ds(base, C)], xis)
            pltpu.sync_copy(dst_hbm.at[pl.ds(base, C)], xid)
            cpa = pltpu.async_copy(nf2_hbm.at[xis], ra0, sga0)
            cpb = pltpu.async_copy(nf2_hbm.at[xid], rb0, sgb0)
            cpa.wait()
            cpb.wait()
            add_rows(ra0, rb0)
            pltpu.sync_copy(ra0, out_hbm.at[pl.ds(base, C)])

    return sc_kernel(nf2, src, dst)


def _edge_body(prev_ref, s_ref, xt_ref, ab_ref, sm_ref, bnd_ref, gbt_ref,
               o_ref, *, blk, blk_off):
    del prev_ref
    xt = xt_ref[...]
    lt = sm_ref[0:8, :]
    m8r = sm_ref[8:9, :]
    ucr = sm_ref[9:10, :]
    bbar = sm_ref[10, 0]
    ccst = sm_ref[10, 1]

    q_t = jnp.dot(lt, xt, preferred_element_type=F32)
    varg_t = (jnp.sum(q_t * q_t, axis=0, keepdims=True)
              + 2.0 * jnp.dot(ucr, xt, preferred_element_type=F32) + ccst)
    inv_t = lax.rsqrt(varg_t + EPS)
    musg_t = (jnp.dot(m8r, xt, preferred_element_type=F32) + bbar) * inv_t
    xs = xt * inv_t
    ones_r = jnp.full((1, blk), 1.0, F32)
    xa = jnp.concatenate([xs, inv_t, musg_t, ones_r], axis=0)

    y0 = s_ref[...] + lax.dot_general(
        xa, ab_ref[0:11, :], (((0,), (0,)), ((), ())),
        preferred_element_type=F32)

    ones128 = jnp.full((128, 1), 1.0 / 128.0, F32)
    m = jnp.dot(y0, ones128, preferred_element_type=F32)
    sq = jnp.dot(y0 * y0, ones128, preferred_element_type=F32)
    rstd = lax.rsqrt(sq - m * m + EPS)

    i = pl.program_id(0) + blk_off
    gidx = i * blk + lax.broadcasted_iota(jnp.int32, (blk, 1), 0)
    starts = bnd_ref[0:1, :]
    ends = bnd_ref[1:2, :]
    oh = jnp.logical_and(gidx >= starts, gidx < ends).astype(F32)
    gb = jnp.dot(oh, gbt_ref[...], preferred_element_type=F32)
    o_ref[...] = jnp.maximum((y0 - m) * (rstd * gb[:, :128]) + gb[:, 128:],
                             0.0)


def _edge_fuse_chunk(prev, S, geo_t, abr, sm, bounds, gbt, *, n_edges,
                     blk, blk_off, first):
    grid = S.shape[0] // blk
    return pl.pallas_call(
        functools.partial(_edge_body, blk=blk, blk_off=blk_off),
        grid=(grid,),
        in_specs=[
            pl.BlockSpec(memory_space=pl.ANY),
            pl.BlockSpec((blk, 128), lambda i: (i, 0)),
            pl.BlockSpec((8, blk), lambda i: (0, i + blk_off)),
            pl.BlockSpec((16, 128), lambda i: (0, 0)),
            pl.BlockSpec((16, 8), lambda i: (0, 0)),
            pl.BlockSpec((8, 16), lambda i: (0, 0)),
            pl.BlockSpec((16, 256), lambda i: (0, 0)),
        ],
        out_specs=pl.BlockSpec((blk, 128), lambda i: (i + blk_off, 0)),
        out_shape=jax.ShapeDtypeStruct((n_edges, 128), F32),
        input_output_aliases=({} if first else {0: 0}),
    )(prev, S, geo_t, abr, sm, bounds, gbt)


def kernel(node_feats, edge_index, edge_geo, cond, batch_ids,
           W_np, b_np, g_np, be_np,
           W_geo, b_geo, g_geo, be_geo,
           Wc1, bc1, g_c, be_c, Wc2, bc2,
           Wf, bf):
    n_edges = edge_index.shape[1]
    src = edge_index[0].astype(jnp.int32)
    dst = edge_index[1].astype(jnp.int32)
    geo_t = edge_geo.T
    bids = batch_ids.astype(jnp.int32)
    starts = jnp.searchsorted(bids, jnp.arange(16, dtype=jnp.int32)).astype(jnp.int32)
    ends = jnp.concatenate([starts[1:], jnp.array([n_edges], jnp.int32)])
    bounds = jnp.concatenate(
        [starts[None], ends[None], jnp.zeros((6, 16), jnp.int32)], axis=0)

    Wf1 = Wf[:128]
    Wf2 = Wf[128:]

    Wgsum = W_geo.reshape(8, 8, 128).sum(axis=0)
    A = Wgsum @ (g_geo[:, None] * Wf2)
    c1 = (b_geo * g_geo) @ Wf2
    vrow = g_geo @ Wf2
    crow = be_geo @ Wf2 + bf
    m8 = Wgsum.mean(axis=1)
    bbar = b_geo.mean()
    acen = Wgsum - m8[:, None]
    bcen = b_geo - bbar
    Mc = (acen @ acen.T) / 128.0
    uc = (acen @ bcen) / 128.0
    ccst = jnp.dot(bcen, bcen) / 128.0
    L = jnp.linalg.cholesky(Mc + 1e-12 * jnp.eye(8, dtype=F32))

    abr = jnp.concatenate(
        [A, c1[None], -vrow[None], crow[None], jnp.zeros((4, 128), F32)],
        axis=0)
    misc8 = jnp.zeros((8,), F32).at[0].set(bbar).at[1].set(ccst)
    sm = jnp.concatenate(
        [L.T, m8[None], uc[None], misc8[None], jnp.zeros((5, 8), F32)],
        axis=0)

    nf2 = _node_proj(node_feats, W_np, Wf1, b_np, g_np, be_np)
    gbt = _cond_proj(cond, Wc1, bc1, g_c, be_c, Wc2, bc2)

    nch = 4
    blk = 3200
    ch = n_edges // nch
    out = jnp.zeros((8, 128), F32)
    for k in range(nch):
        s_k = _sc_gather_sum(nf2, src[k * ch:(k + 1) * ch],
                             dst[k * ch:(k + 1) * ch], ch)
        out = _edge_fuse_chunk(out, s_k, geo_t, abr, sm, bounds, gbt,
                               n_edges=n_edges, blk=blk,
                               blk_off=k * (ch // blk), first=(k == 0))
    return out


# speedup vs baseline: 2.9112x; 1.0152x over previous
import functools

import jax
import jax.numpy as jnp
from jax import lax
from jax.experimental import pallas as pl
from jax.experimental.pallas import tpu as pltpu
from jax.experimental.pallas import tpu_sc as plsc

F32 = jnp.float32
EPS = 1e-5

NC, NS = 2, 16
NW = NC * NS


def _node_proj_body(x_ref, w_ref, wf1_ref, v_ref, o_ref):
    h = jnp.dot(x_ref[...], w_ref[...], preferred_element_type=F32)
    h = h + v_ref[0:1, :]
    m = jnp.mean(h, axis=-1, keepdims=True)
    hc = h - m
    var = jnp.mean(hc * hc, axis=-1, keepdims=True)
    y = hc * lax.rsqrt(var + EPS) * v_ref[1:2, :] + v_ref[2:3, :]
    o_ref[...] = jnp.dot(y, wf1_ref[...], preferred_element_type=F32)


def _node_proj(node_feats, W_np, Wf1, b_np, g_np, be_np):
    n, k = node_feats.shape
    blk = 2000
    grid = n // blk
    vecs = jnp.concatenate(
        [b_np[None], g_np[None], be_np[None], jnp.zeros((5, 128), F32)], axis=0
    )
    return pl.pallas_call(
        _node_proj_body,
        grid=(grid,),
        in_specs=[
            pl.BlockSpec((blk, k), lambda i: (i, 0)),
            pl.BlockSpec((k, 128), lambda i: (0, 0)),
            pl.BlockSpec((128, 128), lambda i: (0, 0)),
            pl.BlockSpec((8, 128), lambda i: (0, 0)),
        ],
        out_specs=pl.BlockSpec((blk, 128), lambda i: (i, 0)),
        out_shape=jax.ShapeDtypeStruct((n, 128), F32),
    )(node_feats, W_np, Wf1, vecs)


def _cond_proj_body(c_ref, w1_ref, w2_ref, v_ref, b2_ref, o_ref):
    h = jnp.dot(c_ref[...], w1_ref[...], preferred_element_type=F32)
    h = h + v_ref[0:1, :]
    m = jnp.mean(h, axis=-1, keepdims=True)
    hc = h - m
    var = jnp.mean(hc * hc, axis=-1, keepdims=True)
    y = hc * lax.rsqrt(var + EPS) * v_ref[1:2, :] + v_ref[2:3, :]
    gb = jnp.dot(y, w2_ref[...], preferred_element_type=F32) + b2_ref[0:1, :]
    lane = lax.broadcasted_iota(jnp.int32, gb.shape, 1)
    o_ref[...] = gb + (lane < 128).astype(F32)


def _cond_proj(cond, Wc1, bc1, g_c, be_c, Wc2, bc2):
    vecs = jnp.concatenate(
        [bc1[None], g_c[None], be_c[None], jnp.zeros((5, 128), F32)], axis=0
    )
    b2 = jnp.concatenate([bc2[None], jnp.zeros((7, 256), F32)], axis=0)
    return pl.pallas_call(
        _cond_proj_body,
        grid=(1,),
        in_specs=[
            pl.BlockSpec((16, 128), lambda i: (0, 0)),
            pl.BlockSpec((128, 128), lambda i: (0, 0)),
            pl.BlockSpec((128, 256), lambda i: (0, 0)),
            pl.BlockSpec((8, 128), lambda i: (0, 0)),
            pl.BlockSpec((8, 256), lambda i: (0, 0)),
        ],
        out_specs=pl.BlockSpec((16, 256), lambda i: (0, 0)),
        out_shape=jax.ShapeDtypeStruct((16, 256), F32),
    )(cond, Wc1, Wc2, vecs, b2)


def _sc_gather_sum(nf2, src, dst, n_edges):
    C = 128
    n_groups = n_edges // C
    base_g = n_groups // NW
    extra = n_groups - base_g * NW
    per_w = base_g * C
    half = base_g // 2
    odd = base_g - 2 * half

    mesh = plsc.VectorSubcoreMesh(
        core_axis_name="c", subcore_axis_name="s", num_cores=NC, num_subcores=NS
    )

    @functools.partial(
        pl.kernel,
        out_type=jax.ShapeDtypeStruct((n_edges, 128), F32),
        mesh=mesh,
        scratch_types=[
            pltpu.VMEM((per_w,), jnp.int32),
            pltpu.VMEM((per_w,), jnp.int32),
            pltpu.VMEM((C, 128), F32),
            pltpu.VMEM((C, 128), F32),
            pltpu.VMEM((C, 128), F32),
            pltpu.VMEM((C, 128), F32),
            pltpu.VMEM((C, 128), F32),
            pltpu.VMEM((C, 128), F32),
            pltpu.VMEM((C,), jnp.int32),
            pltpu.VMEM((C,), jnp.int32),
            pltpu.SemaphoreType.DMA,
            pltpu.SemaphoreType.DMA,
            pltpu.SemaphoreType.DMA,
            pltpu.SemaphoreType.DMA,
            pltpu.SemaphoreType.DMA,
            pltpu.SemaphoreType.DMA,
        ],
    )
    def sc_kernel(nf2_hbm, src_hbm, dst_hbm, out_hbm,
                  ixs, ixd, ra0, rb0, ra1, rb1, rc0, rc1, xis, xid,
                  sga0, sgb0, sga1, sgb1, swb0, swb1):
        wid = lax.axis_index("s") * NC + lax.axis_index("c")
        e0 = wid * per_w
        pltpu.sync_copy(src_hbm.at[pl.ds(e0, per_w)], ixs)
        pltpu.sync_copy(dst_hbm.at[pl.ds(e0, per_w)], ixd)

        def start_gathers(i, ra, rb, sga, sgb):
            sl = pl.ds(i * C, C)
            pltpu.async_copy(nf2_hbm.at[ixs.at[sl]], ra, sga)
            pltpu.async_copy(nf2_hbm.at[ixd.at[sl]], rb, sgb)

        def wait_gathers(ra, rb, sga, sgb):
            pltpu.make_async_copy(nf2_hbm.at[ixs.at[pl.ds(0, C)]], ra, sga).wait()
            pltpu.make_async_copy(nf2_hbm.at[ixd.at[pl.ds(0, C)]], rb, sgb).wait()

        def add_rows(ra, rb, rc):
            def add_body(r, c2):
                for cc in range(8):
                    sl = pl.ds(cc * 16, 16)
                    rc[r, sl] = ra[r, sl] + rb[r, sl]
                return c2
            lax.fori_loop(0, C, add_body, 0)

        def start_wb(i, rc, swb):
            pltpu.async_copy(rc, out_hbm.at[pl.ds(e0 + i * C, C)], swb)

        def wait_wb(rc, swb):
            pltpu.make_async_copy(rc, out_hbm.at[pl.ds(e0, C)], swb).wait()

        def phase(g, j, last_issue, ra, rb, rc, sga, sgb, swb):
            wait_gathers(ra, rb, sga, sgb)

            @pl.when(j > 0)
            def _():
                wait_wb(rc, swb)
            add_rows(ra, rb, rc)
            start_wb(g, rc, swb)

            @pl.when(last_issue)
            def _():
                start_gathers(g + 2, ra, rb, sga, sgb)

        start_gathers(0, ra0, rb0, sga0, sgb0)
        if base_g > 1:
            start_gathers(1, ra1, rb1, sga1, sgb1)

        def body(j, carry):
            g0 = 2 * j
            phase(g0, j, g0 + 2 < base_g, ra0, rb0, rc0, sga0, sgb0, swb0)
            phase(g0 + 1, j, g0 + 3 < base_g, ra1, rb1, rc1, sga1, sgb1,
                  swb1)
            return carry

        if half > 0:
            lax.fori_loop(0, half, body, 0)
        if odd:
            wait_gathers(ra0, rb0, sga0, sgb0)
            if half > 0:
                wait_wb(rc0, swb0)
            add_rows(ra0, rb0, rc0)
            start_wb(base_g - 1, rc0, swb0)
        wait_wb(rc0, swb0)
        if base_g > 1:
            wait_wb(rc1, swb1)

        @pl.when(wid < extra)
        def _():
            base = (NW * base_g + wid) * C
            pltpu.sync_copy(src_hbm.at[pl.ds(base, C)], xis)
            pltpu.sync_copy(dst_hbm.at[pl.ds(base, C)], xid)
            cpa = pltpu.async_copy(nf2_hbm.at[xis], ra0, sga0)
            cpb = pltpu.async_copy(nf2_hbm.at[xid], rb0, sgb0)
            cpa.wait()
            cpb.wait()
            add_rows(ra0, rb0, rc0)
            pltpu.sync_copy(rc0, out_hbm.at[pl.ds(base, C)])

    return sc_kernel(nf2, src, dst)


def _edge_body(prev_ref, s_ref, xt_ref, ab_ref, sm_ref, bnd_ref, gbt_ref,
               o_ref, *, blk, blk_off):
    del prev_ref
    xt = xt_ref[...]
    lt = sm_ref[0:8, :]
    m8r = sm_ref[8:9, :]
    ucr = sm_ref[9:10, :]
    bbar = sm_ref[10, 0]
    ccst = sm_ref[10, 1]

    q_t = jnp.dot(lt, xt, preferred_element_type=F32)
    varg_t = (jnp.sum(q_t * q_t, axis=0, keepdims=True)
              + 2.0 * jnp.dot(ucr, xt, preferred_element_type=F32) + ccst)
    inv_t = lax.rsqrt(varg_t + EPS)
    musg_t = (jnp.dot(m8r, xt, preferred_element_type=F32) + bbar) * inv_t
    xs = xt * inv_t
    ones_r = jnp.full((1, blk), 1.0, F32)
    xa = jnp.concatenate([xs, inv_t, musg_t, ones_r], axis=0)

    y0 = s_ref[...] + lax.dot_general(
        xa, ab_ref[0:11, :], (((0,), (0,)), ((), ())),
        preferred_element_type=F32)

    ones128 = jnp.full((128, 1), 1.0 / 128.0, F32)
    m = jnp.dot(y0, ones128, preferred_element_type=F32)
    sq = jnp.dot(y0 * y0, ones128, preferred_element_type=F32)
    rstd = lax.rsqrt(sq - m * m + EPS)

    i = pl.program_id(0) + blk_off
    gidx = i * blk + lax.broadcasted_iota(jnp.int32, (blk, 1), 0)
    starts = bnd_ref[0:1, :]
    ends = bnd_ref[1:2, :]
    oh = jnp.logical_and(gidx >= starts, gidx < ends).astype(F32)
    gb = jnp.dot(oh, gbt_ref[...], preferred_element_type=F32)
    o_ref[...] = jnp.maximum((y0 - m) * (rstd * gb[:, :128]) + gb[:, 128:],
                             0.0)


def _edge_fuse_chunk(prev, S, geo_t, abr, sm, bounds, gbt, *, n_edges,
                     blk, blk_off, first):
    grid = S.shape[0] // blk
    return pl.pallas_call(
        functools.partial(_edge_body, blk=blk, blk_off=blk_off),
        grid=(grid,),
        in_specs=[
            pl.BlockSpec(memory_space=pl.ANY),
            pl.BlockSpec((blk, 128), lambda i: (i, 0)),
            pl.BlockSpec((8, blk), lambda i: (0, i + blk_off)),
            pl.BlockSpec((16, 128), lambda i: (0, 0)),
            pl.BlockSpec((16, 8), lambda i: (0, 0)),
            pl.BlockSpec((8, 16), lambda i: (0, 0)),
            pl.BlockSpec((16, 256), lambda i: (0, 0)),
        ],
        out_specs=pl.BlockSpec((blk, 128), lambda i: (i + blk_off, 0)),
        out_shape=jax.ShapeDtypeStruct((n_edges, 128), F32),
        input_output_aliases=({} if first else {0: 0}),
    )(prev, S, geo_t, abr, sm, bounds, gbt)


def kernel(node_feats, edge_index, edge_geo, cond, batch_ids,
           W_np, b_np, g_np, be_np,
           W_geo, b_geo, g_geo, be_geo,
           Wc1, bc1, g_c, be_c, Wc2, bc2,
           Wf, bf):
    n_edges = edge_index.shape[1]
    src = edge_index[0].astype(jnp.int32)
    dst = edge_index[1].astype(jnp.int32)
    geo_t = edge_geo.T
    bids = batch_ids.astype(jnp.int32)
    starts = jnp.searchsorted(bids, jnp.arange(16, dtype=jnp.int32)).astype(jnp.int32)
    ends = jnp.concatenate([starts[1:], jnp.array([n_edges], jnp.int32)])
    bounds = jnp.concatenate(
        [starts[None], ends[None], jnp.zeros((6, 16), jnp.int32)], axis=0)

    Wf1 = Wf[:128]
    Wf2 = Wf[128:]

    Wgsum = W_geo.reshape(8, 8, 128).sum(axis=0)
    A = Wgsum @ (g_geo[:, None] * Wf2)
    c1 = (b_geo * g_geo) @ Wf2
    vrow = g_geo @ Wf2
    crow = be_geo @ Wf2 + bf
    m8 = Wgsum.mean(axis=1)
    bbar = b_geo.mean()
    acen = Wgsum - m8[:, None]
    bcen = b_geo - bbar
    Mc = (acen @ acen.T) / 128.0
    uc = (acen @ bcen) / 128.0
    ccst = jnp.dot(bcen, bcen) / 128.0
    L = jnp.linalg.cholesky(Mc + 1e-12 * jnp.eye(8, dtype=F32))

    abr = jnp.concatenate(
        [A, c1[None], -vrow[None], crow[None], jnp.zeros((4, 128), F32)],
        axis=0)
    misc8 = jnp.zeros((8,), F32).at[0].set(bbar).at[1].set(ccst)
    sm = jnp.concatenate(
        [L.T, m8[None], uc[None], misc8[None], jnp.zeros((5, 8), F32)],
        axis=0)

    nf2 = _node_proj(node_feats, W_np, Wf1, b_np, g_np, be_np)
    gbt = _cond_proj(cond, Wc1, bc1, g_c, be_c, Wc2, bc2)

    nch = 4
    blk = 3200
    ch = n_edges // nch
    out = jnp.zeros((8, 128), F32)
    for k in range(nch):
        s_k = _sc_gather_sum(nf2, src[k * ch:(k + 1) * ch],
                             dst[k * ch:(k + 1) * ch], ch)
        out = _edge_fuse_chunk(out, s_k, geo_t, abr, sm, bounds, gbt,
                               n_edges=n_edges, blk=blk,
                               blk_off=k * (ch // blk), first=(k == 0))
    return out
